# baseline (speedup 1.0000x reference)
import jax
import jax.numpy as jnp
from jax import lax
from jax.experimental import pallas as pl
from jax.experimental.pallas import tpu as pltpu
from jax.experimental.pallas import tpu_sc as plsc

XS = 256
NPIX = XS * XS
B = 8
N = 12000
F = 64
FC = 16
NQ = F // FC
NC = 2
NS = 16
L = 16
BPC = B // NC
OWN = NPIX // NS
OWNX = OWN // XS
TRASH = OWN
FCH = 1536
NFC = 8
CAP = 12288


def _sc_body(pil16, cf_hbm, ct_hbm, out0_hbm, out1_hbm,
             acc, bounce, idxbuf, cfc_v, ctc_v, sidx, gbl, sem):
    cid = lax.axis_index("c")
    wid = lax.axis_index("s")
    lane = lax.iota(jnp.int32, L)
    zero16 = jnp.zeros((L,), jnp.float32)
    zero16i = jnp.zeros((L,), jnp.int32)
    trash16 = jnp.full((L,), TRASH, jnp.int32)
    base_lo = OWN * wid

    def _batch(bi, carry):
        bg = cid * BPC + bi

        off = jnp.int32(0)
        for fc in range(NFC):
            p0c = min(FCH * fc, N - FCH)
            pltpu.sync_copy(cf_hbm.at[bg, pl.ds(3 * p0c, 3 * FCH)], cfc_v)
            pltpu.sync_copy(ct_hbm.at[bg, pl.ds(p0c, FCH)], ctc_v)

            def _chunk16(t, off, p0c=p0c, fc=fc):
                gi = lane + t * L
                c1 = plsc.load_gather(cfc_v, [gi * 3 + 1])
                c2 = plsc.load_gather(cfc_v, [gi * 3 + 2])
                ct = ctc_v[pl.ds(t * L, L)]
                local = c1 * XS + c2 - base_lo
                gp = p0c + gi
                keep = ((ct == 1) & (local >= 0) & (local < OWN)
                        & (gp >= FCH * fc))
                plsc.store_compressed(sidx.at[pl.ds(off, L)], local,
                                      mask=keep)
                plsc.store_compressed(gbl.at[pl.ds(off, L)],
                                      (gp + bg * N) * NQ, mask=keep)
                cnt = plsc.all_reduce_population_count(keep)
                return off + jnp.max(cnt)

            off = lax.fori_loop(0, FCH // L, _chunk16, off)
        n_w = off

        w0 = (n_w // L) * L
        live = lane < n_w - w0
        gbl[pl.ds(w0, L)] = jnp.where(live, gbl[pl.ds(w0, L)], 0)
        sidx[pl.ds(w0, L)] = jnp.where(live, sidx[pl.ds(w0, L)], trash16)
        for k in range(1, 9):
            gbl[pl.ds(w0 + k * L, L)] = zero16i
            sidx[pl.ds(w0 + k * L, L)] = trash16

        nch = (n_w + 127) >> 7

        def _fchunk(q, carry):
            def _zero(i, c):
                for f in range(FC):
                    acc[f, i >> 4, pl.ds((i & 15) * L, L)] = zero16
                return c
            lax.fori_loop(0, 256, _zero, 0)

            def _build_start(k, par):
                for s in range(8):
                    idxbuf[par, pl.ds(s * L, L)] = (
                        gbl[pl.ds(k * 128 + s * L, L)] + q)
                pltpu.async_copy(pil16.at[idxbuf.at[par]],
                                 bounce.at[pl.ds(par * 128, 128)], sem)

            @pl.when(nch > 0)
            def _prologue():
                _build_start(jnp.int32(0), jnp.int32(0))

            def _qloop(k, carry):
                par = k & 1
                pltpu.make_async_copy(
                    pil16.at[idxbuf.at[par]],
                    bounce.at[pl.ds(par * 128, 128)], sem).wait()

                @pl.when(k + 1 < nch)
                def _next():
                    _build_start(k + 1, 1 - par)

                def _group(g, c):
                    sv = sidx[pl.ds(k * 128 + g * L, L)]
                    xlv = sv >> 8
                    yv = sv & 255
                    for rr in range(L):
                        vals = bounce[par * 128 + g * L + rr, :]
                        xl = jnp.full((L,), xlv[rr], jnp.int32)
                        y = jnp.full((L,), yv[rr], jnp.int32)
                        plsc.addupdate_scatter(acc, [lane, xl, y], vals)
                    return c
                lax.fori_loop(0, 8, _group, 0)
                return carry
            lax.fori_loop(0, nch, _qloop, 0)

            @pl.when(cid == 0)
            def _co0():
                pltpu.sync_copy(
                    acc.at[:, pl.ds(0, OWNX), :],
                    out0_hbm.at[bi, pl.ds(q * FC, FC),
                                pl.ds(wid * OWNX, OWNX), :])

            @pl.when(cid == 1)
            def _co1():
                pltpu.sync_copy(
                    acc.at[:, pl.ds(0, OWNX), :],
                    out1_hbm.at[bi, pl.ds(q * FC, FC),
                                pl.ds(wid * OWNX, OWNX), :])
            return carry
        lax.fori_loop(0, NQ, _fchunk, 0)
        return carry

    lax.fori_loop(0, BPC, _batch, 0)


def _sc_scatter(pil16, coord_flat, contains):
    mesh = plsc.VectorSubcoreMesh(core_axis_name="c", subcore_axis_name="s",
                                  num_cores=NC, num_subcores=NS)
    return pl.kernel(
        _sc_body,
        out_type=[jax.ShapeDtypeStruct((BPC, F, XS, XS), jnp.float32),
                  jax.ShapeDtypeStruct((BPC, F, XS, XS), jnp.float32)],
        mesh=mesh,
        compiler_params=pltpu.CompilerParams(use_tc_tiling_on_sc=False,
                                             needs_layout_passes=False),
        scratch_types=[
            pltpu.VMEM((FC, OWNX + 1, XS), jnp.float32),
            pltpu.VMEM((256, FC), jnp.float32),
            pltpu.VMEM((2, 128), jnp.int32),
            pltpu.VMEM((3 * FCH,), jnp.int32),
            pltpu.VMEM((FCH,), jnp.int32),
            pltpu.VMEM((CAP,), jnp.int32),
            pltpu.VMEM((CAP,), jnp.int32),
            pltpu.SemaphoreType.DMA,
        ],
    )(pil16, coord_flat, contains)


def kernel(pillars, coord, contains_pillars):
    pil16 = pillars.reshape(B * N * NQ, FC)
    coord_flat = coord.reshape(B, N * 3)
    out0, out1 = _sc_scatter(pil16, coord_flat, contains_pillars)
    return jnp.concatenate([out0, out1], axis=0)


# speedup vs baseline: 1.3058x; 1.3058x over previous
import jax
import jax.numpy as jnp
from jax import lax
from jax.experimental import pallas as pl
from jax.experimental.pallas import tpu as pltpu
from jax.experimental.pallas import tpu_sc as plsc

XS = 256
NPIX = XS * XS
B = 8
N = 12000
F = 64
FC = 16
NQ = F // FC
NC = 2
NS = 16
L = 16
BPC = B // NC
OWN = NPIX // NS
OWNX = OWN // XS
TRASH = OWN
FCH = 2048
NFC = 6
CAP = 12288
NBUF = 8


def _sc_body(pil16, cf_hbm, ct_hbm, out_hbm,
             acc, bounce, idxbuf, cfc_v, ctc_v, sidx, gbl,
             gsem, osem, fsem):
    cid = lax.axis_index("c")
    wid = lax.axis_index("s")
    lane = lax.iota(jnp.int32, L)
    zero16 = jnp.zeros((L,), jnp.float32)
    zero16i = jnp.zeros((L,), jnp.int32)
    trash16 = jnp.full((L,), TRASH, jnp.int32)
    base_lo = OWN * wid

    def _stage(bg, c, par):
        p0c = min(FCH * c, N - FCH)
        pltpu.async_copy(cf_hbm.at[bg, pl.ds(3 * p0c, 3 * FCH)],
                         cfc_v.at[par], fsem)
        pltpu.async_copy(ct_hbm.at[bg, pl.ds(p0c, FCH)],
                         ctc_v.at[par], fsem)

    def _stage_wait(bg, c, par):
        p0c = min(FCH * c, N - FCH)
        pltpu.make_async_copy(cf_hbm.at[bg, pl.ds(3 * p0c, 3 * FCH)],
                              cfc_v.at[par], fsem).wait()
        pltpu.make_async_copy(ct_hbm.at[bg, pl.ds(p0c, FCH)],
                              ctc_v.at[par], fsem).wait()

    def _batch(bi, carry):
        bg = cid * BPC + bi

        _stage(bg, 0, 0)
        off = jnp.int32(0)
        for c in range(NFC):
            par = c & 1
            p0c = min(FCH * c, N - FCH)
            _stage_wait(bg, c, par)
            if c + 1 < NFC:
                _stage(bg, c + 1, 1 - par)

            def _chunk16(t, off, p0c=p0c, c=c, par=par):
                gi = lane + t * L
                c1 = plsc.load_gather(cfc_v.at[par], [gi * 3 + 1])
                c2 = plsc.load_gather(cfc_v.at[par], [gi * 3 + 2])
                ct = ctc_v[par, pl.ds(t * L, L)]
                local = c1 * XS + c2 - base_lo
                gp = p0c + gi
                keep = ((ct == 1) & (local >= 0) & (local < OWN)
                        & (gp >= FCH * c))
                plsc.store_compressed(sidx.at[pl.ds(off, L)], local,
                                      mask=keep)
                plsc.store_compressed(gbl.at[pl.ds(off, L)],
                                      (gp + bg * N) * NQ, mask=keep)
                cnt = plsc.all_reduce_population_count(keep)
                return off + jnp.max(cnt)

            off = lax.fori_loop(0, FCH // L, _chunk16, off)
        n_w = off

        w0 = (n_w // L) * L
        live = lane < n_w - w0
        gbl[pl.ds(w0, L)] = jnp.where(live, gbl[pl.ds(w0, L)], 0)
        sidx[pl.ds(w0, L)] = jnp.where(live, sidx[pl.ds(w0, L)], trash16)
        for k in range(1, 9):
            gbl[pl.ds(w0 + k * L, L)] = zero16i
            sidx[pl.ds(w0 + k * L, L)] = trash16

        nch = (n_w + 127) >> 7

        def _fchunk(q, carry):
            u = bi * NQ + q

            def _fire(k, c):
                par = k & (NBUF - 1)
                for s in range(8):
                    idxbuf[par, pl.ds(s * L, L)] = (
                        gbl[pl.ds(k * 128 + s * L, L)] + q)
                pltpu.async_copy(pil16.at[idxbuf.at[par]],
                                 bounce.at[pl.ds(par * 128, 128)], gsem)
                return c

            lax.fori_loop(0, jnp.minimum(NBUF, nch), _fire, 0)

            @pl.when(u > 0)
            def _drain_copyout():
                pltpu.make_async_copy(
                    acc.at[:, pl.ds(0, OWNX), :],
                    out_hbm.at[bg, pl.ds(q * FC, FC),
                               pl.ds(wid * OWNX, OWNX), :], osem).wait()

            def _zero(i, c):
                for f in range(FC):
                    acc[f, i >> 4, pl.ds((i & 15) * L, L)] = zero16
                return c
            lax.fori_loop(0, 256, _zero, 0)

            def _qloop(k, carry):
                par = k & (NBUF - 1)
                pltpu.make_async_copy(
                    pil16.at[idxbuf.at[par]],
                    bounce.at[pl.ds(par * 128, 128)], gsem).wait()

                @pl.when(k + NBUF < nch)
                def _next():
                    _fire(k + NBUF, 0)

                def _group(g, c):
                    sv = sidx[pl.ds(k * 128 + g * L, L)]
                    xlv = sv >> 8
                    yv = sv & 255
                    for rr in range(L):
                        vals = bounce[par * 128 + g * L + rr, :]
                        xl = jnp.full((L,), xlv[rr], jnp.int32)
                        y = jnp.full((L,), yv[rr], jnp.int32)
                        plsc.addupdate_scatter(acc, [lane, xl, y], vals)
                    return c
                lax.fori_loop(0, 8, _group, 0)
                return carry
            lax.fori_loop(0, nch, _qloop, 0)

            pltpu.async_copy(
                acc.at[:, pl.ds(0, OWNX), :],
                out_hbm.at[bg, pl.ds(q * FC, FC), pl.ds(wid * OWNX, OWNX), :],
                osem)
            return carry
        lax.fori_loop(0, NQ, _fchunk, 0)
        return carry

    lax.fori_loop(0, BPC, _batch, 0)

    pltpu.make_async_copy(
        acc.at[:, pl.ds(0, OWNX), :],
        out_hbm.at[0, pl.ds(0, FC), pl.ds(wid * OWNX, OWNX), :], osem).wait()


def _sc_scatter(pil16, coord_flat, contains):
    mesh = plsc.VectorSubcoreMesh(core_axis_name="c", subcore_axis_name="s",
                                  num_cores=NC, num_subcores=NS)
    return pl.kernel(
        _sc_body,
        out_type=jax.ShapeDtypeStruct((B, F, XS, XS), jnp.float32),
        mesh=mesh,
        compiler_params=pltpu.CompilerParams(use_tc_tiling_on_sc=False,
                                             needs_layout_passes=False),
        scratch_types=[
            pltpu.VMEM((FC, OWNX + 1, XS), jnp.float32),
            pltpu.VMEM((NBUF * 128, FC), jnp.float32),
            pltpu.VMEM((NBUF, 128), jnp.int32),
            pltpu.VMEM((2, 3 * FCH), jnp.int32),
            pltpu.VMEM((2, FCH), jnp.int32),
            pltpu.VMEM((CAP,), jnp.int32),
            pltpu.VMEM((CAP,), jnp.int32),
            pltpu.SemaphoreType.DMA,
            pltpu.SemaphoreType.DMA,
            pltpu.SemaphoreType.DMA,
        ],
    )(pil16, coord_flat, contains)


def kernel(pillars, coord, contains_pillars):
    pil16 = pillars.reshape(B * N * NQ, FC)
    coord_flat = coord.reshape(B, N * 3)
    return _sc_scatter(pil16, coord_flat, contains_pillars)


# speedup vs baseline: 1.3110x; 1.0040x over previous
import jax
import jax.numpy as jnp
from jax import lax
from jax.experimental import pallas as pl
from jax.experimental.pallas import tpu as pltpu
from jax.experimental.pallas import tpu_sc as plsc

XS = 256
NPIX = XS * XS
B = 8
N = 12000
F = 64
FC = 16
NQ = F // FC
NC = 2
NS = 16
L = 16
BPC = B // NC
OWN = NPIX // NS
OWNX = OWN // XS
TRASH = OWN
FCH = 2048
NFC = 6
CAP = 12288
NBUF = 8


def _sc_body(pil16, cf_hbm, ct_hbm, out_hbm,
             acc, bounce, idxbuf, cfc_v, ctc_v, sidx, gbl,
             gsem, osem, fsem):
    cid = lax.axis_index("c")
    wid = lax.axis_index("s")
    lane = lax.iota(jnp.int32, L)
    zero16 = jnp.zeros((L,), jnp.float32)
    zero16i = jnp.zeros((L,), jnp.int32)
    trash16 = jnp.full((L,), TRASH, jnp.int32)
    base_lo = OWN * wid

    def _stage(bg, c, par):
        p0c = min(FCH * c, N - FCH)
        pltpu.async_copy(cf_hbm.at[bg, pl.ds(3 * p0c, 3 * FCH)],
                         cfc_v.at[par], fsem)
        pltpu.async_copy(ct_hbm.at[bg, pl.ds(p0c, FCH)],
                         ctc_v.at[par], fsem)

    def _stage_wait(bg, c, par):
        p0c = min(FCH * c, N - FCH)
        pltpu.make_async_copy(cf_hbm.at[bg, pl.ds(3 * p0c, 3 * FCH)],
                              cfc_v.at[par], fsem).wait()
        pltpu.make_async_copy(ct_hbm.at[bg, pl.ds(p0c, FCH)],
                              ctc_v.at[par], fsem).wait()

    def _batch(bi, carry):
        bg = cid * BPC + bi

        _stage(bg, 0, 0)
        off = jnp.int32(0)
        for c in range(NFC):
            par = c & 1
            p0c = min(FCH * c, N - FCH)
            _stage_wait(bg, c, par)
            if c + 1 < NFC:
                _stage(bg, c + 1, 1 - par)

            def _chunk16(t, off, p0c=p0c, c=c, par=par):
                gi = lane + t * L
                c1 = plsc.load_gather(cfc_v.at[par], [gi * 3 + 1])
                c2 = plsc.load_gather(cfc_v.at[par], [gi * 3 + 2])
                ct = ctc_v[par, pl.ds(t * L, L)]
                local = c1 * XS + c2 - base_lo
                gp = p0c + gi
                keep = ((ct == 1) & (local >= 0) & (local < OWN)
                        & (gp >= FCH * c))
                plsc.store_compressed(sidx.at[pl.ds(off, L)], local,
                                      mask=keep)
                plsc.store_compressed(gbl.at[pl.ds(off, L)],
                                      (gp + bg * N) * NQ, mask=keep)
                cnt = plsc.all_reduce_population_count(keep)
                return off + jnp.max(cnt)

            off = lax.fori_loop(0, FCH // L, _chunk16, off)
        n_w = off

        w0 = (n_w // L) * L
        live = lane < n_w - w0
        gbl[pl.ds(w0, L)] = jnp.where(live, gbl[pl.ds(w0, L)], 0)
        sidx[pl.ds(w0, L)] = jnp.where(live, sidx[pl.ds(w0, L)], trash16)
        for k in range(1, 9):
            gbl[pl.ds(w0 + k * L, L)] = zero16i
            sidx[pl.ds(w0 + k * L, L)] = trash16

        nch = (n_w + 127) >> 7

        def _fchunk(q, carry):
            u = bi * NQ + q

            def _fire(k, c):
                par = k & (NBUF - 1)
                for s in range(8):
                    idxbuf[par, pl.ds(s * L, L)] = (
                        gbl[pl.ds(k * 128 + s * L, L)] + q)
                pltpu.async_copy(pil16.at[idxbuf.at[par]],
                                 bounce.at[pl.ds(par * 128, 128)], gsem)
                return c

            lax.fori_loop(0, jnp.minimum(NBUF, nch), _fire, 0)

            @pl.when(u > 0)
            def _drain_copyout():
                pltpu.make_async_copy(
                    acc.at[:, pl.ds(0, OWN)],
                    out_hbm.at[bg, pl.ds(q * FC, FC),
                               pl.ds(wid * OWN, OWN)], osem).wait()

            def _zero(i, c):
                for f in range(FC):
                    acc[f, pl.ds(i * L, L)] = zero16
                return c
            lax.fori_loop(0, 256, _zero, 0)

            def _qloop(k, carry):
                par = k & (NBUF - 1)
                pltpu.make_async_copy(
                    pil16.at[idxbuf.at[par]],
                    bounce.at[pl.ds(par * 128, 128)], gsem).wait()

                @pl.when(k + NBUF < nch)
                def _next():
                    _fire(k + NBUF, 0)

                def _group(g, c):
                    sv = sidx[pl.ds(k * 128 + g * L, L)]
                    for rr in range(L):
                        vals = bounce[par * 128 + g * L + rr, :]
                        sid = jnp.full((L,), sv[rr], jnp.int32)
                        plsc.addupdate_scatter(acc, [lane, sid], vals)
                    return c
                lax.fori_loop(0, 8, _group, 0)
                return carry
            lax.fori_loop(0, nch, _qloop, 0)

            pltpu.async_copy(
                acc.at[:, pl.ds(0, OWN)],
                out_hbm.at[bg, pl.ds(q * FC, FC), pl.ds(wid * OWN, OWN)],
                osem)
            return carry
        lax.fori_loop(0, NQ, _fchunk, 0)
        return carry

    lax.fori_loop(0, BPC, _batch, 0)

    pltpu.make_async_copy(
        acc.at[:, pl.ds(0, OWN)],
        out_hbm.at[0, pl.ds(0, FC), pl.ds(wid * OWN, OWN)], osem).wait()


def _sc_scatter(pil16, coord_flat, contains):
    mesh = plsc.VectorSubcoreMesh(core_axis_name="c", subcore_axis_name="s",
                                  num_cores=NC, num_subcores=NS)
    return pl.kernel(
        _sc_body,
        out_type=jax.ShapeDtypeStruct((B, F, NPIX), jnp.float32),
        mesh=mesh,
        compiler_params=pltpu.CompilerParams(use_tc_tiling_on_sc=False,
                                             needs_layout_passes=False),
        scratch_types=[
            pltpu.VMEM((FC, OWN + XS), jnp.float32),
            pltpu.VMEM((NBUF * 128, FC), jnp.float32),
            pltpu.VMEM((NBUF, 128), jnp.int32),
            pltpu.VMEM((2, 3 * FCH), jnp.int32),
            pltpu.VMEM((2, FCH), jnp.int32),
            pltpu.VMEM((CAP,), jnp.int32),
            pltpu.VMEM((CAP,), jnp.int32),
            pltpu.SemaphoreType.DMA,
            pltpu.SemaphoreType.DMA,
            pltpu.SemaphoreType.DMA,
        ],
    )(pil16, coord_flat, contains)


def kernel(pillars, coord, contains_pillars):
    pil16 = pillars.reshape(B * N * NQ, FC)
    coord_flat = coord.reshape(B, N * 3)
    out3 = _sc_scatter(pil16, coord_flat, contains_pillars)
    return out3.reshape(B, F, XS, XS)
